# baseline (speedup 1.0000x reference)
import jax
import jax.numpy as jnp
from jax import lax
from jax.experimental import pallas as pl

S = 197
K = 12
H = 12
NEG_INF = float("-inf")


def _dot(a, b, dims):
    return lax.dot_general(a, b, (dims, ((), ())),
                           preferred_element_type=jnp.float32)


def _body(x_ref, r3_ref, out_ref):
    P = [_dot(x_ref[0, 0, h], x_ref[1, 0, h], ((1,), (1,)))
         for h in range(H)]
    Q = [_dot(P[h], x_ref[2, 0, h], ((1,), (0,)))
         for h in range(H)]
    u4 = [_dot(r3_ref[0, h, 0:1], Q[h], ((1,), (1,))) for h in range(H)]
    u4t = [_dot(Q[h], r3_ref[0, h], ((1,), (1,)))[:, 0:1] for h in range(H)]

    col_i = lax.broadcasted_iota(jnp.int32, (S, 1), 0)
    row_j = lax.broadcasted_iota(jnp.int32, (1, S), 1)
    one = jnp.float32(1.0)
    zero = jnp.float32(0.0)
    lo = jnp.where(row_j < col_i, one, zero)
    kio = lax.broadcasted_iota(jnp.int32, (1, K), 1).astype(jnp.float32)
    ivc = (col_i - 1).astype(jnp.float32)
    ones_col = jnp.ones((S, 1), jnp.float32)

    Cs = []
    for h in range(H):
        s_row = jnp.where(row_j == 0, NEG_INF, u4[h])
        s_col = jnp.where(col_i == 0, NEG_INF, u4t[h])
        Cs.append(jnp.where(s_row > s_col, one,
                            jnp.where(s_row == s_col, lo, zero)))
    ranks = [_dot(Cs[h], ones_col, ((1,), (0,))) for h in range(H)]
    for h in range(H):
        onehot = jnp.where(ranks[h] == kio, one, zero)
        out_f = jnp.sum(onehot * ivc, axis=0, keepdims=True)
        out_ref[0, h] = out_f.astype(jnp.int32)


@jax.jit
def kernel(x):
    L, B, nh, s1, s2 = x.shape
    assert (nh, s1, s2) == (H, S, S)

    r3 = jnp.broadcast_to(x[3, :, :, 0:1, :], (B, nh, 8, S))

    grid = (B,)
    out = pl.pallas_call(
        _body,
        grid=grid,
        in_specs=[
            pl.BlockSpec((3, 1, H, S, S), lambda b: (0, b, 0, 0, 0)),
            pl.BlockSpec((1, H, 8, S), lambda b: (b, 0, 0, 0)),
        ],
        out_specs=pl.BlockSpec((1, H, 1, K), lambda b: (b, 0, 0, 0)),
        out_shape=jax.ShapeDtypeStruct((B, H, 1, K), jnp.int32),
    )(x, r3)
    return out.reshape(B, H, K)


# speedup vs baseline: 1.1109x; 1.1109x over previous
import jax
import jax.numpy as jnp
from jax import lax
from jax.experimental import pallas as pl

S = 197
K = 12
H = 12
NEG_INF = float("-inf")


def _dot(a, b, dims):
    return lax.dot_general(a, b, (dims, ((), ())),
                           preferred_element_type=jnp.float32)


def _body(x_ref, r3_ref, out_ref):
    P = [_dot(x_ref[0, 0, h], x_ref[1, 0, h], ((1,), (1,)))
         for h in range(H)]
    Q = [_dot(P[h].astype(jnp.bfloat16), x_ref[2, 0, h], ((1,), (0,)))
         for h in range(H)]
    u4 = [_dot(r3_ref[0, h, 0:1], Q[h], ((1,), (1,))) for h in range(H)]
    u4t = [_dot(Q[h], r3_ref[0, h], ((1,), (1,)))[:, 0:1] for h in range(H)]

    col_i = lax.broadcasted_iota(jnp.int32, (S, 1), 0)
    row_j = lax.broadcasted_iota(jnp.int32, (1, S), 1)
    one = jnp.float32(1.0)
    zero = jnp.float32(0.0)
    lo = jnp.where(row_j < col_i, one, zero)
    kio = lax.broadcasted_iota(jnp.int32, (1, K), 1).astype(jnp.float32)
    ivc = (col_i - 1).astype(jnp.float32)
    ones_col = jnp.ones((S, 1), jnp.float32)

    Cs = []
    for h in range(H):
        s_row = jnp.where(row_j == 0, NEG_INF, u4[h])
        s_col = jnp.where(col_i == 0, NEG_INF, u4t[h])
        Cs.append(jnp.where(s_row > s_col, one,
                            jnp.where(s_row == s_col, lo, zero)))
    ranks = [_dot(Cs[h], ones_col, ((1,), (0,))) for h in range(H)]
    for h in range(H):
        onehot = jnp.where(ranks[h] == kio, one, zero)
        out_f = jnp.sum(onehot * ivc, axis=0, keepdims=True)
        out_ref[0, h] = out_f.astype(jnp.int32)


@jax.jit
def kernel(x):
    L, B, nh, s1, s2 = x.shape
    assert (nh, s1, s2) == (H, S, S)

    xb = x[0:3].astype(jnp.bfloat16)
    r3 = jnp.broadcast_to(x[3, :, :, 0:1, :], (B, nh, 8, S))

    grid = (B,)
    out = pl.pallas_call(
        _body,
        grid=grid,
        in_specs=[
            pl.BlockSpec((3, 1, H, S, S), lambda b: (0, b, 0, 0, 0)),
            pl.BlockSpec((1, H, 8, S), lambda b: (b, 0, 0, 0)),
        ],
        out_specs=pl.BlockSpec((1, H, 1, K), lambda b: (b, 0, 0, 0)),
        out_shape=jax.ShapeDtypeStruct((B, H, 1, K), jnp.int32),
    )(xb, r3)
    return out.reshape(B, H, K)
